# Initial kernel scaffold; baseline (speedup 1.0000x reference)
import jax
import jax.numpy as jnp
from jax.experimental import pallas as pl


def kernel(token_ids, weight):
    raise NotImplementedError("write your pallas kernel here")


# speedup vs baseline: 1.7962x; 1.7962x over previous
import functools

import jax
import jax.numpy as jnp
from jax import lax
from jax.experimental import pallas as pl
from jax.experimental.pallas import tpu as pltpu
from jax.experimental.pallas import tpu_sc as plsc

_NUM_CORES = 2
_NUM_SUBCORES = 16
_NUM_WORKERS = _NUM_CORES * _NUM_SUBCORES

_CHUNK = 512


def _gather_body(ids_hbm, table_hbm, out_hbm, idx_v, rows_v, sem):
    b_per_w = ids_hbm.shape[0] // _NUM_WORKERS
    wid = lax.axis_index("s") * _NUM_CORES + lax.axis_index("c")
    base = wid * b_per_w
    num_chunks = b_per_w // _CHUNK

    def body(i, carry):
        off = base + i * _CHUNK
        pltpu.sync_copy(ids_hbm.at[pl.ds(off, _CHUNK)], idx_v)
        pltpu.async_copy(table_hbm.at[idx_v], rows_v, sem).wait()
        pltpu.sync_copy(rows_v, out_hbm.at[pl.ds(off, _CHUNK)])
        return carry

    lax.fori_loop(0, num_chunks, body, 0)


@jax.jit
def kernel(token_ids, weight):
    orig_shape = token_ids.shape
    flat_ids = token_ids.reshape(-1).astype(jnp.int32)
    n = flat_ids.shape[0]
    dim = weight.shape[1]

    mesh = plsc.VectorSubcoreMesh(
        core_axis_name="c",
        subcore_axis_name="s",
        num_cores=_NUM_CORES,
        num_subcores=_NUM_SUBCORES,
    )
    run = pl.kernel(
        _gather_body,
        out_type=jax.ShapeDtypeStruct((n, dim), weight.dtype),
        mesh=mesh,
        scratch_types=[
            pltpu.VMEM((_CHUNK,), jnp.int32),
            pltpu.VMEM((_CHUNK, dim), weight.dtype),
            pltpu.SemaphoreType.DMA,
        ],
        compiler_params=pltpu.CompilerParams(use_tc_tiling_on_sc=False),
    )
    out = run(flat_ids, weight)
    return out.reshape(*orig_shape, dim)


# speedup vs baseline: 1.8718x; 1.0420x over previous
import functools

import jax
import jax.numpy as jnp
from jax import lax
from jax.experimental import pallas as pl
from jax.experimental.pallas import tpu as pltpu
from jax.experimental.pallas import tpu_sc as plsc

_NUM_CORES = 2
_NUM_SUBCORES = 16
_NUM_WORKERS = _NUM_CORES * _NUM_SUBCORES

_CHUNK = 800


def _gather_body(ids_hbm, table_hbm, out_hbm, idx_v, rows_v, sems_i, sems_g, sems_o):
    n_ids = ids_hbm.shape[0]
    b_per_w = n_ids // _NUM_WORKERS
    wid = lax.axis_index("s") * _NUM_CORES + lax.axis_index("c")
    base = wid * b_per_w
    nchunks = b_per_w // _CHUNK

    def idx_start(i, b):
        pltpu.async_copy(ids_hbm.at[pl.ds(base + i * _CHUNK, _CHUNK)],
                         idx_v.at[b], sems_i[b])

    def idx_wait(b):
        pltpu.make_async_copy(ids_hbm.at[pl.ds(base, _CHUNK)],
                              idx_v.at[b], sems_i[b]).wait()

    def gather_start(b):
        pltpu.async_copy(table_hbm.at[idx_v.at[b]], rows_v.at[b], sems_g[b])

    def gather_wait(b):
        pltpu.make_async_copy(table_hbm.at[idx_v.at[b]],
                              rows_v.at[b], sems_g[b]).wait()

    def out_start(i, b):
        pltpu.async_copy(rows_v.at[b],
                         out_hbm.at[pl.ds(base + i * _CHUNK, _CHUNK)], sems_o[b])

    def out_wait(b):
        pltpu.make_async_copy(rows_v.at[b],
                              out_hbm.at[pl.ds(base, _CHUNK)], sems_o[b]).wait()

    idx_start(0, 0)
    idx_start(1, 1)
    idx_wait(0)
    gather_start(0)

    def group(g, carry):
        for b in (0, 1):
            i = 2 * g + b
            o = 1 - b
            gather_wait(b)
            out_start(i, b)

            @pl.when(i + 2 < nchunks)
            def _():
                idx_start(i + 2, b)

            @pl.when(i + 1 < nchunks)
            def _():
                idx_wait(o)

                @pl.when(i >= 1)
                def _():
                    out_wait(o)

                gather_start(o)
        return carry

    lax.fori_loop(0, nchunks // 2, group, 0)

    out_wait(0)
    out_wait(1)


@jax.jit
def kernel(token_ids, weight):
    orig_shape = token_ids.shape
    flat_ids = token_ids.reshape(-1).astype(jnp.int32)
    n = flat_ids.shape[0]
    dim = weight.shape[1]

    mesh = plsc.VectorSubcoreMesh(
        core_axis_name="c",
        subcore_axis_name="s",
        num_cores=_NUM_CORES,
        num_subcores=_NUM_SUBCORES,
    )
    run = pl.kernel(
        _gather_body,
        out_type=jax.ShapeDtypeStruct((n, dim), weight.dtype),
        mesh=mesh,
        scratch_types=[
            pltpu.VMEM((2, _CHUNK), jnp.int32),
            pltpu.VMEM((2, _CHUNK, dim), weight.dtype),
            (pltpu.SemaphoreType.DMA, pltpu.SemaphoreType.DMA),
            (pltpu.SemaphoreType.DMA, pltpu.SemaphoreType.DMA),
            (pltpu.SemaphoreType.DMA, pltpu.SemaphoreType.DMA),
        ],
        compiler_params=pltpu.CompilerParams(use_tc_tiling_on_sc=False),
    )
    out = run(flat_ids, weight)
    return out.reshape(*orig_shape, dim)


# speedup vs baseline: 1.8747x; 1.0016x over previous
import functools

import jax
import jax.numpy as jnp
from jax import lax
from jax.experimental import pallas as pl
from jax.experimental.pallas import tpu as pltpu
from jax.experimental.pallas import tpu_sc as plsc

_NUM_CORES = 2
_NUM_SUBCORES = 16
_NUM_WORKERS = _NUM_CORES * _NUM_SUBCORES

_CHUNK = 800


def _gather_body(ids_hbm, table_hbm, out_hbm, idx_v, rows_v, sems_i, sems_g, sems_o):
    n_ids = ids_hbm.shape[0]
    b_per_w = n_ids // _NUM_WORKERS
    wid = lax.axis_index("s") * _NUM_CORES + lax.axis_index("c")
    base = wid * b_per_w
    nchunks = b_per_w // _CHUNK

    def idx_start(i, b):
        pltpu.async_copy(ids_hbm.at[pl.ds(base + i * _CHUNK, _CHUNK)],
                         idx_v.at[b], sems_i[b])

    def idx_wait(b):
        pltpu.make_async_copy(ids_hbm.at[pl.ds(base, _CHUNK)],
                              idx_v.at[b], sems_i[b]).wait()

    def gather_start(b):
        pltpu.async_copy(table_hbm.at[idx_v.at[b]], rows_v.at[b], sems_g[b])

    def gather_wait(b):
        pltpu.make_async_copy(table_hbm.at[idx_v.at[b]],
                              rows_v.at[b], sems_g[b]).wait()

    def out_start(i, b):
        pltpu.async_copy(rows_v.at[b],
                         out_hbm.at[pl.ds(base + i * _CHUNK, _CHUNK)], sems_o[b])

    def out_wait(b):
        pltpu.make_async_copy(rows_v.at[b],
                              out_hbm.at[pl.ds(base, _CHUNK)], sems_o[b]).wait()

    idx_start(0, 0)
    idx_start(1, 1)
    idx_wait(0)
    gather_start(0)

    def group(g, carry):
        for b in (0, 1):
            i = 2 * g + b
            o = 1 - b

            @pl.when(i + 1 < nchunks)
            def _():
                idx_wait(o)

                @pl.when(i >= 1)
                def _():
                    out_wait(o)

                gather_start(o)

            gather_wait(b)
            out_start(i, b)

            @pl.when(i + 2 < nchunks)
            def _():
                idx_start(i + 2, b)
        return carry

    lax.fori_loop(0, nchunks // 2, group, 0)

    out_wait(0)
    out_wait(1)


@jax.jit
def kernel(token_ids, weight):
    orig_shape = token_ids.shape
    flat_ids = token_ids.reshape(-1).astype(jnp.int32)
    n = flat_ids.shape[0]
    dim = weight.shape[1]

    mesh = plsc.VectorSubcoreMesh(
        core_axis_name="c",
        subcore_axis_name="s",
        num_cores=_NUM_CORES,
        num_subcores=_NUM_SUBCORES,
    )
    run = pl.kernel(
        _gather_body,
        out_type=jax.ShapeDtypeStruct((n, dim), weight.dtype),
        mesh=mesh,
        scratch_types=[
            pltpu.VMEM((2, _CHUNK), jnp.int32),
            pltpu.VMEM((2, _CHUNK, dim), weight.dtype),
            (pltpu.SemaphoreType.DMA, pltpu.SemaphoreType.DMA),
            (pltpu.SemaphoreType.DMA, pltpu.SemaphoreType.DMA),
            (pltpu.SemaphoreType.DMA, pltpu.SemaphoreType.DMA),
        ],
        compiler_params=pltpu.CompilerParams(use_tc_tiling_on_sc=False),
    )
    out = run(flat_ids, weight)
    return out.reshape(*orig_shape, dim)
